# speedup vs baseline: 23.6454x; 23.6454x over previous
import jax
import jax.numpy as jnp
from jax.experimental import pallas as pl
from jax.experimental.pallas import tpu as pltpu

_N_THR = 10


def _body(pr_ref, tr_ref, pd_ref, td_ref, pr_out, rc_out, de_out):
    p = pr_ref[...]
    t = tr_ref[...]
    dx = td_ref[0] - pd_ref[0]
    dy = td_ref[1] - pd_ref[1]
    dist = jnp.sqrt(dx * dx + dy * dy)
    sig = jax.nn.sigmoid(p)

    prec_acc = jnp.zeros((p.shape[0], 1), jnp.float32)
    rec_acc = jnp.zeros((p.shape[0], 1), jnp.float32)
    derr_acc = jnp.zeros((p.shape[0], 1), jnp.float32)
    for i in range(_N_THR):
        thr = jnp.float32(0.1 * i + 0.05)
        pos = sig > thr
        tru = t > thr
        both = jnp.logical_and(tru, pos)
        bothf = both.astype(jnp.float32)
        tp = jnp.sum(bothf, axis=1, keepdims=True)
        npos = jnp.sum(pos.astype(jnp.float32), axis=1, keepdims=True)
        ntru = jnp.sum(tru.astype(jnp.float32), axis=1, keepdims=True)
        dsum = jnp.sum(dist * bothf, axis=1, keepdims=True)
        prec_acc = prec_acc + tp / (npos + 1.0)
        rec_acc = rec_acc + tp / (ntru + 1.0)
        derr_acc = derr_acc + dsum / jnp.maximum(tp, 1.0)

    inv = 1.0 / (_N_THR * p.shape[0])
    pr_out[0, 0] = jnp.sum(prec_acc) * inv
    rc_out[0, 0] = jnp.sum(rec_acc) * inv
    de_out[0, 0] = jnp.sum(derr_acc) * inv


def kernel(pred_rank, pred_disp_vector, target_rank, target_disp_vector, anchors):
    del anchors
    pd = jnp.transpose(pred_disp_vector, (2, 0, 1))
    td = jnp.transpose(target_disp_vector, (2, 0, 1))
    out = pl.pallas_call(
        _body,
        out_shape=(
            jax.ShapeDtypeStruct((1, 1), jnp.float32),
            jax.ShapeDtypeStruct((1, 1), jnp.float32),
            jax.ShapeDtypeStruct((1, 1), jnp.float32),
        ),
        out_specs=(
            pl.BlockSpec(memory_space=pltpu.SMEM),
            pl.BlockSpec(memory_space=pltpu.SMEM),
            pl.BlockSpec(memory_space=pltpu.SMEM),
        ),
    )(pred_rank, target_rank, pd, td)
    return (out[0][0, 0], out[1][0, 0], out[2][0, 0])
